# baseline (speedup 1.0000x reference)
import functools
import jax
import jax.numpy as jnp
from jax import lax
from jax.experimental import pallas as pl
from jax.experimental.pallas import tpu as pltpu
from jax.experimental.pallas import tpu_sc as plsc

N = 10000
E = 320000
D = 128
NUM_CLASSES = 10
NUM_GRAPHS = 64

NC = 2
NS = 16
NW = NC * NS
K = 128
CPT = 80
EPT = K * CPT
EPAD = NW * EPT
NACC = 10112
STRIPE = NACC // NS

NCH = 2560
CPW = NCH // NW
WCH = CPW // 2

R = 1000
GRID = N // R

_sc_mesh = functools.partial(
    plsc.VectorSubcoreMesh, core_axis_name="c", subcore_axis_name="s",
    num_cores=NC, num_subcores=NS)


def _sc_count1_body(idx_hbm, ones_hbm, zeros_hbm, out_hbm,
                    idxv, onesv, acc, csem):
    cid = lax.axis_index("c")
    sid = lax.axis_index("s")
    wid = cid * NS + sid
    pltpu.sync_copy(idx_hbm.at[wid], idxv)
    pltpu.sync_copy(ones_hbm, onesv)
    pltpu.sync_copy(zeros_hbm, acc.at[pl.ds(sid * STRIPE, STRIPE)])
    plsc.subcore_barrier()

    def csfire(c):
        pltpu.async_copy(onesv, acc.at[idxv.at[c]], csem, add=True)

    def csdrain(c):
        pltpu.make_async_copy(onesv, acc.at[idxv.at[c]], csem).wait()

    csfire(0)

    def pairc(t, carry):
        c = 2 * t
        csfire(c + 1)
        csdrain(c)
        csfire(c + 2)
        csdrain(c + 1)
        return carry

    lax.fori_loop(0, CPT // 2 - 1, pairc, 0)
    cl = CPT - 2
    csfire(cl + 1)
    csdrain(cl)
    csdrain(cl + 1)
    plsc.subcore_barrier()
    sl = pl.ds(sid * STRIPE, STRIPE)
    pltpu.sync_copy(acc.at[sl], out_hbm.at[cid, sl])


def _sc_count1(idx_p, ones_a, zeros_w):
    return pl.kernel(
        _sc_count1_body,
        out_type=jax.ShapeDtypeStruct((NC, NACC, D), jnp.float32),
        mesh=_sc_mesh(),
        scratch_types=[
            pltpu.VMEM((CPT, K), jnp.int32),
            pltpu.VMEM((K, D), jnp.float32),
            pltpu.VMEM_SHARED((NACC, D), jnp.float32),
            pltpu.SemaphoreType.DMA,
        ],
    )(idx_p, ones_a, zeros_w)


def _sc_segsum_body(mt_hbm, src_hbm, dst_hbm, out_hbm,
                    srcv, dstv, r0, r1, acc, gsem):
    cid = lax.axis_index("c")
    sid = lax.axis_index("s")
    wid = cid * NS + sid

    def zrow(r, carry):
        for j in range(D // 16):
            r0.at[r][pl.ds(j * 16, 16)] = jnp.zeros((16,), jnp.float32)
        return carry

    lax.fori_loop(0, K, zrow, 0)
    base = sid * STRIPE
    for j in range(STRIPE // K):
        pltpu.sync_copy(r0, acc.at[pl.ds(base + j * K, K)])
    remn = STRIPE - (STRIPE // K) * K
    if remn:
        pltpu.sync_copy(r0.at[pl.ds(0, remn)],
                        acc.at[pl.ds(base + (STRIPE // K) * K, remn)])
    plsc.subcore_barrier()

    def fire(c, buf):
        pltpu.async_copy(mt_hbm.at[srcv.at[c]], buf, gsem)

    def drain(c, buf):
        pltpu.make_async_copy(mt_hbm.at[srcv.at[c]], buf, gsem).wait()

    def pair(t, carry):
        c = 2 * t
        drain(c, r0)
        fire(c + 1, r1)
        pltpu.sync_copy(r0, acc.at[dstv.at[c]], add=True)
        drain(c + 1, r1)
        fire(c + 2, r0)
        pltpu.sync_copy(r1, acc.at[dstv.at[c + 1]], add=True)
        return carry

    for h in range(CPW // WCH):
        start = cid * NS * CPW + sid * CPW + h * WCH
        pltpu.sync_copy(src_hbm.at[pl.ds(start, WCH)], srcv)
        pltpu.sync_copy(dst_hbm.at[pl.ds(start, WCH)], dstv)
        fire(0, r0)
        lax.fori_loop(0, WCH // 2 - 1, pair, 0)
        cl = WCH - 2
        drain(cl, r0)
        fire(cl + 1, r1)
        pltpu.sync_copy(r0, acc.at[dstv.at[cl]], add=True)
        drain(cl + 1, r1)
        pltpu.sync_copy(r1, acc.at[dstv.at[cl + 1]], add=True)
    plsc.subcore_barrier()
    sl = pl.ds(sid * STRIPE, STRIPE)
    pltpu.sync_copy(acc.at[sl], out_hbm.at[cid, sl])


def _sc_segsum(mt, src_f, dst_f):
    return pl.kernel(
        _sc_segsum_body,
        out_type=jax.ShapeDtypeStruct((NC, NACC, D), jnp.float32),
        mesh=_sc_mesh(),
        scratch_types=[
            pltpu.VMEM((WCH, K), jnp.int32),
            pltpu.VMEM((WCH, K), jnp.int32),
            pltpu.VMEM((K, D), jnp.float32),
            pltpu.VMEM((K, D), jnp.float32),
            pltpu.VMEM_SHARED((NACC, D), jnp.float32),
            pltpu.SemaphoreType.DMA,
        ],
    )(mt, src_f, dst_f)


def _tc_first_body(emb_ref, w_ref, dinv_ref, m_ref, mt_ref):
    m = jnp.dot(emb_ref[...], w_ref[...], preferred_element_type=jnp.float32)
    m_ref[...] = m
    mt_ref[...] = m * dinv_ref[...]


def _tc_first(emb, w, dinv):
    return pl.pallas_call(
        _tc_first_body,
        grid=(GRID,),
        in_specs=[
            pl.BlockSpec((R, D), lambda i: (i, 0)),
            pl.BlockSpec((D, D), lambda i: (0, 0)),
            pl.BlockSpec((R, 1), lambda i: (i, 0)),
        ],
        out_specs=[
            pl.BlockSpec((R, D), lambda i: (i, 0)),
            pl.BlockSpec((R, D), lambda i: (i, 0)),
        ],
        out_shape=[jax.ShapeDtypeStruct((N, D), jnp.float32),
                   jax.ShapeDtypeStruct((N, D), jnp.float32)],
    )(emb, w, dinv)


def _tc_mid_body(parts_ref, mprev_ref, dinv_ref, b_ref, w_ref, m_ref, mt_ref):
    di = dinv_ref[...]
    p = parts_ref[...]
    h = di * (p[0] + p[1]) + (di * di) * mprev_ref[...] + b_ref[...]
    h = jnp.maximum(h, 0.0)
    m = jnp.dot(h, w_ref[...], preferred_element_type=jnp.float32)
    m_ref[...] = m
    mt_ref[...] = m * di


def _tc_mid(parts, mprev, dinv, b, w):
    return pl.pallas_call(
        _tc_mid_body,
        grid=(GRID,),
        in_specs=[
            pl.BlockSpec((NC, R, D), lambda i: (0, i, 0)),
            pl.BlockSpec((R, D), lambda i: (i, 0)),
            pl.BlockSpec((R, 1), lambda i: (i, 0)),
            pl.BlockSpec((1, D), lambda i: (0, 0)),
            pl.BlockSpec((D, D), lambda i: (0, 0)),
        ],
        out_specs=[
            pl.BlockSpec((R, D), lambda i: (i, 0)),
            pl.BlockSpec((R, D), lambda i: (i, 0)),
        ],
        out_shape=[jax.ShapeDtypeStruct((N, D), jnp.float32),
                   jax.ShapeDtypeStruct((N, D), jnp.float32)],
    )(parts, mprev, dinv, b, w)


def _tc_pool_body(parts_ref, m3_ref, dinv_ref, b_ref, batch_ref, mcnt_ref,
                  linw_ref, linb_ref, out_ref, sums, cnt):
    i = pl.program_id(0)

    @pl.when(i == 0)
    def _init():
        sums[...] = jnp.zeros_like(sums)
        cnt[...] = jnp.zeros_like(cnt)

    di = dinv_ref[...]
    p = parts_ref[...]
    h = di * (p[0] + p[1]) + (di * di) * m3_ref[...] + b_ref[...]
    mf = (mcnt_ref[...] > 0.0).astype(jnp.float32)
    g = lax.broadcasted_iota(jnp.int32, (1, NUM_GRAPHS), 1)
    onehot = (batch_ref[...] == g).astype(jnp.float32)
    hm = h * mf
    dn = (((0,), (0,)), ((), ()))
    sums[...] += lax.dot_general(onehot, hm, dn,
                                 preferred_element_type=jnp.float32)
    cnt[...] += lax.dot_general(onehot, mf, dn,
                                preferred_element_type=jnp.float32)

    @pl.when(i == pl.num_programs(0) - 1)
    def _fin():
        pooled = sums[...] / jnp.maximum(cnt[...], 1.0)
        out_ref[...] = jnp.dot(pooled, linw_ref[...],
                               preferred_element_type=jnp.float32) + linb_ref[...]


def _tc_pool(parts, m3, dinv, b, batch2, mcnt, linw, linb):
    return pl.pallas_call(
        _tc_pool_body,
        grid=(GRID,),
        in_specs=[
            pl.BlockSpec((NC, R, D), lambda i: (0, i, 0)),
            pl.BlockSpec((R, D), lambda i: (i, 0)),
            pl.BlockSpec((R, 1), lambda i: (i, 0)),
            pl.BlockSpec((1, D), lambda i: (0, 0)),
            pl.BlockSpec((R, 1), lambda i: (i, 0)),
            pl.BlockSpec((R, 1), lambda i: (i, 0)),
            pl.BlockSpec((D, NUM_CLASSES), lambda i: (0, 0)),
            pl.BlockSpec((1, NUM_CLASSES), lambda i: (0, 0)),
        ],
        out_specs=pl.BlockSpec((NUM_GRAPHS, NUM_CLASSES), lambda i: (0, 0)),
        out_shape=jax.ShapeDtypeStruct((NUM_GRAPHS, NUM_CLASSES), jnp.float32),
        scratch_shapes=[pltpu.VMEM((NUM_GRAPHS, D), jnp.float32),
                        pltpu.VMEM((NUM_GRAPHS, 1), jnp.float32)],
    )(parts, m3, dinv, b, batch2, mcnt, linw, linb)


@jax.jit
def kernel(x, edge_index, edge_type, batch, emb, W1, b1, W2, b2, W3, b3,
           linW, linb):
    del x, edge_type
    src = edge_index[0]
    dst = edge_index[1]
    pad = EPAD - E
    src_p = jnp.concatenate([src, jnp.full((pad,), N, jnp.int32)]
                            ).reshape(NW, CPT, K)
    dst_p = jnp.concatenate([dst, jnp.full((pad,), N, jnp.int32)]
                            ).reshape(NW, CPT, K)
    padf = NCH * K - E
    pidx = jnp.arange(padf, dtype=jnp.int32)
    src_f = jnp.concatenate([src, pidx % 8192]).reshape(NCH, K)
    dst_f = jnp.concatenate([dst, N + (pidx % (NACC - N))]).reshape(NCH, K)
    col = jnp.arange(D, dtype=jnp.int32)[None, :]
    ones_a = jnp.broadcast_to((col == 0).astype(jnp.float32), (K, D))
    zeros_w = jnp.zeros((STRIPE, D), jnp.float32)

    cnt_s = _sc_count1(src_p, ones_a, zeros_w)
    cnt_d = _sc_count1(dst_p, ones_a, zeros_w)
    csrc = cnt_s[0, :N, 0:1] + cnt_s[1, :N, 0:1]
    cdst = cnt_d[0, :N, 0:1] + cnt_d[1, :N, 0:1]
    dinv = lax.rsqrt(cdst + 1.0)
    mcnt = csrc + cdst

    m1, mt1 = _tc_first(emb, W1, dinv)
    parts1 = _sc_segsum(mt1, src_f, dst_f)
    m2, mt2 = _tc_mid(parts1, m1, dinv, b1.reshape(1, D), W2)
    parts2 = _sc_segsum(mt2, src_f, dst_f)
    m3, mt3 = _tc_mid(parts2, m2, dinv, b2.reshape(1, D), W3)
    parts3 = _sc_segsum(mt3, src_f, dst_f)
    return _tc_pool(parts3, m3, dinv, b3.reshape(1, D),
                    batch.reshape(N, 1), mcnt,
                    linW, linb.reshape(1, NUM_CLASSES))


# speedup vs baseline: 1.0094x; 1.0094x over previous
import functools
import jax
import jax.numpy as jnp
from jax import lax
from jax.experimental import pallas as pl
from jax.experimental.pallas import tpu as pltpu
from jax.experimental.pallas import tpu_sc as plsc

N = 10000
E = 320000
D = 128
NUM_CLASSES = 10
NUM_GRAPHS = 64

NC = 2
NS = 16
NW = NC * NS
K = 128
CPT = 80
EPT = K * CPT
EPAD = NW * EPT
NACC = 10112
STRIPE = NACC // NS

NCH = 2560
CPW = NCH // NW
WCH = CPW // 2

R = 2000
GRID = N // R

_sc_mesh = functools.partial(
    plsc.VectorSubcoreMesh, core_axis_name="c", subcore_axis_name="s",
    num_cores=NC, num_subcores=NS)


def _sc_count1_body(idx_hbm, ones_hbm, zeros_hbm, out_hbm,
                    idxv, onesv, acc, csem):
    cid = lax.axis_index("c")
    sid = lax.axis_index("s")
    wid = cid * NS + sid
    pltpu.sync_copy(idx_hbm.at[wid], idxv)
    pltpu.sync_copy(ones_hbm, onesv)
    pltpu.sync_copy(zeros_hbm, acc.at[pl.ds(sid * STRIPE, STRIPE)])
    plsc.subcore_barrier()

    def csfire(c):
        pltpu.async_copy(onesv, acc.at[idxv.at[c]], csem, add=True)

    def csdrain(c):
        pltpu.make_async_copy(onesv, acc.at[idxv.at[c]], csem).wait()

    csfire(0)

    def pairc(t, carry):
        c = 2 * t
        csfire(c + 1)
        csdrain(c)
        csfire(c + 2)
        csdrain(c + 1)
        return carry

    lax.fori_loop(0, CPT // 2 - 1, pairc, 0)
    cl = CPT - 2
    csfire(cl + 1)
    csdrain(cl)
    csdrain(cl + 1)
    plsc.subcore_barrier()
    sl = pl.ds(sid * STRIPE, STRIPE)
    pltpu.sync_copy(acc.at[sl], out_hbm.at[cid, sl])


def _sc_count1(idx_p, ones_a, zeros_w):
    return pl.kernel(
        _sc_count1_body,
        out_type=jax.ShapeDtypeStruct((NC, NACC, D), jnp.float32),
        mesh=_sc_mesh(),
        scratch_types=[
            pltpu.VMEM((CPT, K), jnp.int32),
            pltpu.VMEM((K, D), jnp.float32),
            pltpu.VMEM_SHARED((NACC, D), jnp.float32),
            pltpu.SemaphoreType.DMA,
        ],
    )(idx_p, ones_a, zeros_w)


def _sc_segsum_body(mt_hbm, src_hbm, dst_hbm, out_hbm,
                    srcv, dstv, r0, r1, acc, gsem):
    cid = lax.axis_index("c")
    sid = lax.axis_index("s")
    wid = cid * NS + sid

    def zrow(r, carry):
        for j in range(D // 16):
            r0.at[r][pl.ds(j * 16, 16)] = jnp.zeros((16,), jnp.float32)
        return carry

    lax.fori_loop(0, K, zrow, 0)
    base = sid * STRIPE
    for j in range(STRIPE // K):
        pltpu.sync_copy(r0, acc.at[pl.ds(base + j * K, K)])
    remn = STRIPE - (STRIPE // K) * K
    if remn:
        pltpu.sync_copy(r0.at[pl.ds(0, remn)],
                        acc.at[pl.ds(base + (STRIPE // K) * K, remn)])
    plsc.subcore_barrier()

    def fire(c, buf):
        pltpu.async_copy(mt_hbm.at[srcv.at[c]], buf, gsem)

    def drain(c, buf):
        pltpu.make_async_copy(mt_hbm.at[srcv.at[c]], buf, gsem).wait()

    def pair(t, carry):
        c = 2 * t
        drain(c, r0)
        fire(c + 1, r1)
        pltpu.sync_copy(r0, acc.at[dstv.at[c]], add=True)
        drain(c + 1, r1)
        fire(c + 2, r0)
        pltpu.sync_copy(r1, acc.at[dstv.at[c + 1]], add=True)
        return carry

    for h in range(CPW // WCH):
        start = cid * NS * CPW + sid * CPW + h * WCH
        pltpu.sync_copy(src_hbm.at[pl.ds(start, WCH)], srcv)
        pltpu.sync_copy(dst_hbm.at[pl.ds(start, WCH)], dstv)
        fire(0, r0)
        lax.fori_loop(0, WCH // 2 - 1, pair, 0)
        cl = WCH - 2
        drain(cl, r0)
        fire(cl + 1, r1)
        pltpu.sync_copy(r0, acc.at[dstv.at[cl]], add=True)
        drain(cl + 1, r1)
        pltpu.sync_copy(r1, acc.at[dstv.at[cl + 1]], add=True)
    plsc.subcore_barrier()
    sl = pl.ds(sid * STRIPE, STRIPE)
    pltpu.sync_copy(acc.at[sl], out_hbm.at[cid, sl])


def _sc_segsum(mt, src_f, dst_f):
    return pl.kernel(
        _sc_segsum_body,
        out_type=jax.ShapeDtypeStruct((NC, NACC, D), jnp.float32),
        mesh=_sc_mesh(),
        scratch_types=[
            pltpu.VMEM((WCH, K), jnp.int32),
            pltpu.VMEM((WCH, K), jnp.int32),
            pltpu.VMEM((K, D), jnp.float32),
            pltpu.VMEM((K, D), jnp.float32),
            pltpu.VMEM_SHARED((NACC, D), jnp.float32),
            pltpu.SemaphoreType.DMA,
        ],
    )(mt, src_f, dst_f)


def _tc_first_body(emb_ref, w_ref, dinv_ref, m_ref, mt_ref):
    m = jnp.dot(emb_ref[...], w_ref[...], preferred_element_type=jnp.float32)
    m_ref[...] = m
    mt_ref[...] = m * dinv_ref[...]


def _tc_first(emb, w, dinv):
    return pl.pallas_call(
        _tc_first_body,
        grid=(GRID,),
        in_specs=[
            pl.BlockSpec((R, D), lambda i: (i, 0)),
            pl.BlockSpec((D, D), lambda i: (0, 0)),
            pl.BlockSpec((R, 1), lambda i: (i, 0)),
        ],
        out_specs=[
            pl.BlockSpec((R, D), lambda i: (i, 0)),
            pl.BlockSpec((R, D), lambda i: (i, 0)),
        ],
        out_shape=[jax.ShapeDtypeStruct((N, D), jnp.float32),
                   jax.ShapeDtypeStruct((N, D), jnp.float32)],
    )(emb, w, dinv)


def _tc_mid_body(parts_ref, mprev_ref, dinv_ref, b_ref, w_ref, m_ref, mt_ref):
    di = dinv_ref[...]
    p = parts_ref[...]
    h = di * (p[0] + p[1]) + (di * di) * mprev_ref[...] + b_ref[...]
    h = jnp.maximum(h, 0.0)
    m = jnp.dot(h, w_ref[...], preferred_element_type=jnp.float32)
    m_ref[...] = m
    mt_ref[...] = m * di


def _tc_mid(parts, mprev, dinv, b, w):
    return pl.pallas_call(
        _tc_mid_body,
        grid=(GRID,),
        in_specs=[
            pl.BlockSpec((NC, R, D), lambda i: (0, i, 0)),
            pl.BlockSpec((R, D), lambda i: (i, 0)),
            pl.BlockSpec((R, 1), lambda i: (i, 0)),
            pl.BlockSpec((1, D), lambda i: (0, 0)),
            pl.BlockSpec((D, D), lambda i: (0, 0)),
        ],
        out_specs=[
            pl.BlockSpec((R, D), lambda i: (i, 0)),
            pl.BlockSpec((R, D), lambda i: (i, 0)),
        ],
        out_shape=[jax.ShapeDtypeStruct((N, D), jnp.float32),
                   jax.ShapeDtypeStruct((N, D), jnp.float32)],
    )(parts, mprev, dinv, b, w)


def _tc_pool_body(parts_ref, m3_ref, dinv_ref, b_ref, batch_ref, mcnt_ref,
                  linw_ref, linb_ref, out_ref, sums, cnt):
    i = pl.program_id(0)

    @pl.when(i == 0)
    def _init():
        sums[...] = jnp.zeros_like(sums)
        cnt[...] = jnp.zeros_like(cnt)

    di = dinv_ref[...]
    p = parts_ref[...]
    h = di * (p[0] + p[1]) + (di * di) * m3_ref[...] + b_ref[...]
    mf = (mcnt_ref[...] > 0.0).astype(jnp.float32)
    g = lax.broadcasted_iota(jnp.int32, (1, NUM_GRAPHS), 1)
    onehot = (batch_ref[...] == g).astype(jnp.float32)
    hm = h * mf
    dn = (((0,), (0,)), ((), ()))
    sums[...] += lax.dot_general(onehot, hm, dn,
                                 preferred_element_type=jnp.float32)
    cnt[...] += lax.dot_general(onehot, mf, dn,
                                preferred_element_type=jnp.float32)

    @pl.when(i == pl.num_programs(0) - 1)
    def _fin():
        pooled = sums[...] / jnp.maximum(cnt[...], 1.0)
        out_ref[...] = jnp.dot(pooled, linw_ref[...],
                               preferred_element_type=jnp.float32) + linb_ref[...]


def _tc_pool(parts, m3, dinv, b, batch2, mcnt, linw, linb):
    return pl.pallas_call(
        _tc_pool_body,
        grid=(GRID,),
        in_specs=[
            pl.BlockSpec((NC, R, D), lambda i: (0, i, 0)),
            pl.BlockSpec((R, D), lambda i: (i, 0)),
            pl.BlockSpec((R, 1), lambda i: (i, 0)),
            pl.BlockSpec((1, D), lambda i: (0, 0)),
            pl.BlockSpec((R, 1), lambda i: (i, 0)),
            pl.BlockSpec((R, 1), lambda i: (i, 0)),
            pl.BlockSpec((D, NUM_CLASSES), lambda i: (0, 0)),
            pl.BlockSpec((1, NUM_CLASSES), lambda i: (0, 0)),
        ],
        out_specs=pl.BlockSpec((NUM_GRAPHS, NUM_CLASSES), lambda i: (0, 0)),
        out_shape=jax.ShapeDtypeStruct((NUM_GRAPHS, NUM_CLASSES), jnp.float32),
        scratch_shapes=[pltpu.VMEM((NUM_GRAPHS, D), jnp.float32),
                        pltpu.VMEM((NUM_GRAPHS, 1), jnp.float32)],
    )(parts, m3, dinv, b, batch2, mcnt, linw, linb)


@jax.jit
def kernel(x, edge_index, edge_type, batch, emb, W1, b1, W2, b2, W3, b3,
           linW, linb):
    del x, edge_type
    src = edge_index[0]
    dst = edge_index[1]
    pad = EPAD - E
    src_p = jnp.concatenate([src, jnp.full((pad,), N, jnp.int32)]
                            ).reshape(NW, CPT, K)
    dst_p = jnp.concatenate([dst, jnp.full((pad,), N, jnp.int32)]
                            ).reshape(NW, CPT, K)
    padf = NCH * K - E
    pidx = jnp.arange(padf, dtype=jnp.int32)
    src_f = jnp.concatenate([src, pidx % 8192]).reshape(NCH, K)
    dst_f = jnp.concatenate([dst, N + (pidx % (NACC - N))]).reshape(NCH, K)
    col = jnp.arange(D, dtype=jnp.int32)[None, :]
    ones_a = jnp.broadcast_to((col == 0).astype(jnp.float32), (K, D))
    zeros_w = jnp.zeros((STRIPE, D), jnp.float32)

    cnt_s = _sc_count1(src_p, ones_a, zeros_w)
    cnt_d = _sc_count1(dst_p, ones_a, zeros_w)
    csrc = cnt_s[0, :N, 0:1] + cnt_s[1, :N, 0:1]
    cdst = cnt_d[0, :N, 0:1] + cnt_d[1, :N, 0:1]
    dinv = lax.rsqrt(cdst + 1.0)
    mcnt = csrc + cdst

    m1, mt1 = _tc_first(emb, W1, dinv)
    parts1 = _sc_segsum(mt1, src_f, dst_f)
    m2, mt2 = _tc_mid(parts1, m1, dinv, b1.reshape(1, D), W2)
    parts2 = _sc_segsum(mt2, src_f, dst_f)
    m3, mt3 = _tc_mid(parts2, m2, dinv, b2.reshape(1, D), W3)
    parts3 = _sc_segsum(mt3, src_f, dst_f)
    return _tc_pool(parts3, m3, dinv, b3.reshape(1, D),
                    batch.reshape(N, 1), mcnt,
                    linW, linb.reshape(1, NUM_CLASSES))


# speedup vs baseline: 1.0174x; 1.0079x over previous
import functools
import jax
import jax.numpy as jnp
from jax import lax
from jax.experimental import pallas as pl
from jax.experimental.pallas import tpu as pltpu
from jax.experimental.pallas import tpu_sc as plsc

N = 10000
E = 320000
D = 128
NUM_CLASSES = 10
NUM_GRAPHS = 64

NC = 2
NS = 16
NW = NC * NS
K = 128
CPT = 80
EPT = K * CPT
EPAD = NW * EPT
NACC = 10112
STRIPE = NACC // NS

NCH = 2560
CPW = NCH // NW
WCH = CPW // 2

R = 5000
GRID = N // R

_sc_mesh = functools.partial(
    plsc.VectorSubcoreMesh, core_axis_name="c", subcore_axis_name="s",
    num_cores=NC, num_subcores=NS)


def _sc_count1_body(idx_hbm, ones_hbm, zeros_hbm, out_hbm,
                    idxv, onesv, acc, csem):
    cid = lax.axis_index("c")
    sid = lax.axis_index("s")
    wid = cid * NS + sid
    pltpu.sync_copy(idx_hbm.at[wid], idxv)
    pltpu.sync_copy(ones_hbm, onesv)
    pltpu.sync_copy(zeros_hbm, acc.at[pl.ds(sid * STRIPE, STRIPE)])
    plsc.subcore_barrier()

    def csfire(c):
        pltpu.async_copy(onesv, acc.at[idxv.at[c]], csem, add=True)

    def csdrain(c):
        pltpu.make_async_copy(onesv, acc.at[idxv.at[c]], csem).wait()

    csfire(0)

    def pairc(t, carry):
        c = 2 * t
        csfire(c + 1)
        csdrain(c)
        csfire(c + 2)
        csdrain(c + 1)
        return carry

    lax.fori_loop(0, CPT // 2 - 1, pairc, 0)
    cl = CPT - 2
    csfire(cl + 1)
    csdrain(cl)
    csdrain(cl + 1)
    plsc.subcore_barrier()
    sl = pl.ds(sid * STRIPE, STRIPE)
    pltpu.sync_copy(acc.at[sl], out_hbm.at[cid, sl])


def _sc_count1(idx_p, ones_a, zeros_w):
    return pl.kernel(
        _sc_count1_body,
        out_type=jax.ShapeDtypeStruct((NC, NACC, D), jnp.float32),
        mesh=_sc_mesh(),
        scratch_types=[
            pltpu.VMEM((CPT, K), jnp.int32),
            pltpu.VMEM((K, D), jnp.float32),
            pltpu.VMEM_SHARED((NACC, D), jnp.float32),
            pltpu.SemaphoreType.DMA,
        ],
    )(idx_p, ones_a, zeros_w)


def _sc_segsum_body(mt_hbm, src_hbm, dst_hbm, out_hbm,
                    srcv, dstv, r0, r1, acc, gsem):
    cid = lax.axis_index("c")
    sid = lax.axis_index("s")
    wid = cid * NS + sid

    def zrow(r, carry):
        for j in range(D // 16):
            r0.at[r][pl.ds(j * 16, 16)] = jnp.zeros((16,), jnp.float32)
        return carry

    lax.fori_loop(0, K, zrow, 0)
    base = sid * STRIPE
    for j in range(STRIPE // K):
        pltpu.sync_copy(r0, acc.at[pl.ds(base + j * K, K)])
    remn = STRIPE - (STRIPE // K) * K
    if remn:
        pltpu.sync_copy(r0.at[pl.ds(0, remn)],
                        acc.at[pl.ds(base + (STRIPE // K) * K, remn)])
    plsc.subcore_barrier()

    def fire(c, buf):
        pltpu.async_copy(mt_hbm.at[srcv.at[c]], buf, gsem)

    def drain(c, buf):
        pltpu.make_async_copy(mt_hbm.at[srcv.at[c]], buf, gsem).wait()

    def pair(t, carry):
        c = 2 * t
        drain(c, r0)
        fire(c + 1, r1)
        pltpu.sync_copy(r0, acc.at[dstv.at[c]], add=True)
        drain(c + 1, r1)
        fire(c + 2, r0)
        pltpu.sync_copy(r1, acc.at[dstv.at[c + 1]], add=True)
        return carry

    for h in range(CPW // WCH):
        start = cid * NS * CPW + sid * CPW + h * WCH
        pltpu.sync_copy(src_hbm.at[pl.ds(start, WCH)], srcv)
        pltpu.sync_copy(dst_hbm.at[pl.ds(start, WCH)], dstv)
        fire(0, r0)
        lax.fori_loop(0, WCH // 2 - 1, pair, 0)
        cl = WCH - 2
        drain(cl, r0)
        fire(cl + 1, r1)
        pltpu.sync_copy(r0, acc.at[dstv.at[cl]], add=True)
        drain(cl + 1, r1)
        pltpu.sync_copy(r1, acc.at[dstv.at[cl + 1]], add=True)
    plsc.subcore_barrier()
    sl = pl.ds(sid * STRIPE, STRIPE)
    pltpu.sync_copy(acc.at[sl], out_hbm.at[cid, sl])


def _sc_segsum(mt, src_f, dst_f):
    return pl.kernel(
        _sc_segsum_body,
        out_type=jax.ShapeDtypeStruct((NC, NACC, D), jnp.float32),
        mesh=_sc_mesh(),
        scratch_types=[
            pltpu.VMEM((WCH, K), jnp.int32),
            pltpu.VMEM((WCH, K), jnp.int32),
            pltpu.VMEM((K, D), jnp.float32),
            pltpu.VMEM((K, D), jnp.float32),
            pltpu.VMEM_SHARED((NACC, D), jnp.float32),
            pltpu.SemaphoreType.DMA,
        ],
    )(mt, src_f, dst_f)


def _tc_first_body(emb_ref, w_ref, dinv_ref, m_ref, mt_ref):
    m = jnp.dot(emb_ref[...], w_ref[...], preferred_element_type=jnp.float32)
    m_ref[...] = m
    mt_ref[...] = m * dinv_ref[...]


def _tc_first(emb, w, dinv):
    return pl.pallas_call(
        _tc_first_body,
        grid=(GRID,),
        in_specs=[
            pl.BlockSpec((R, D), lambda i: (i, 0)),
            pl.BlockSpec((D, D), lambda i: (0, 0)),
            pl.BlockSpec((R, 1), lambda i: (i, 0)),
        ],
        out_specs=[
            pl.BlockSpec((R, D), lambda i: (i, 0)),
            pl.BlockSpec((R, D), lambda i: (i, 0)),
        ],
        out_shape=[jax.ShapeDtypeStruct((N, D), jnp.float32),
                   jax.ShapeDtypeStruct((N, D), jnp.float32)],
    )(emb, w, dinv)


def _tc_mid_body(parts_ref, mprev_ref, dinv_ref, b_ref, w_ref, m_ref, mt_ref):
    di = dinv_ref[...]
    p = parts_ref[...]
    h = di * (p[0] + p[1]) + (di * di) * mprev_ref[...] + b_ref[...]
    h = jnp.maximum(h, 0.0)
    m = jnp.dot(h, w_ref[...], preferred_element_type=jnp.float32)
    m_ref[...] = m
    mt_ref[...] = m * di


def _tc_mid(parts, mprev, dinv, b, w):
    return pl.pallas_call(
        _tc_mid_body,
        grid=(GRID,),
        in_specs=[
            pl.BlockSpec((NC, R, D), lambda i: (0, i, 0)),
            pl.BlockSpec((R, D), lambda i: (i, 0)),
            pl.BlockSpec((R, 1), lambda i: (i, 0)),
            pl.BlockSpec((1, D), lambda i: (0, 0)),
            pl.BlockSpec((D, D), lambda i: (0, 0)),
        ],
        out_specs=[
            pl.BlockSpec((R, D), lambda i: (i, 0)),
            pl.BlockSpec((R, D), lambda i: (i, 0)),
        ],
        out_shape=[jax.ShapeDtypeStruct((N, D), jnp.float32),
                   jax.ShapeDtypeStruct((N, D), jnp.float32)],
    )(parts, mprev, dinv, b, w)


def _tc_pool_body(parts_ref, m3_ref, dinv_ref, b_ref, batch_ref, mcnt_ref,
                  linw_ref, linb_ref, out_ref, sums, cnt):
    i = pl.program_id(0)

    @pl.when(i == 0)
    def _init():
        sums[...] = jnp.zeros_like(sums)
        cnt[...] = jnp.zeros_like(cnt)

    di = dinv_ref[...]
    p = parts_ref[...]
    h = di * (p[0] + p[1]) + (di * di) * m3_ref[...] + b_ref[...]
    mf = (mcnt_ref[...] > 0.0).astype(jnp.float32)
    g = lax.broadcasted_iota(jnp.int32, (1, NUM_GRAPHS), 1)
    onehot = (batch_ref[...] == g).astype(jnp.float32)
    hm = h * mf
    dn = (((0,), (0,)), ((), ()))
    sums[...] += lax.dot_general(onehot, hm, dn,
                                 preferred_element_type=jnp.float32)
    cnt[...] += lax.dot_general(onehot, mf, dn,
                                preferred_element_type=jnp.float32)

    @pl.when(i == pl.num_programs(0) - 1)
    def _fin():
        pooled = sums[...] / jnp.maximum(cnt[...], 1.0)
        out_ref[...] = jnp.dot(pooled, linw_ref[...],
                               preferred_element_type=jnp.float32) + linb_ref[...]


def _tc_pool(parts, m3, dinv, b, batch2, mcnt, linw, linb):
    return pl.pallas_call(
        _tc_pool_body,
        grid=(GRID,),
        in_specs=[
            pl.BlockSpec((NC, R, D), lambda i: (0, i, 0)),
            pl.BlockSpec((R, D), lambda i: (i, 0)),
            pl.BlockSpec((R, 1), lambda i: (i, 0)),
            pl.BlockSpec((1, D), lambda i: (0, 0)),
            pl.BlockSpec((R, 1), lambda i: (i, 0)),
            pl.BlockSpec((R, 1), lambda i: (i, 0)),
            pl.BlockSpec((D, NUM_CLASSES), lambda i: (0, 0)),
            pl.BlockSpec((1, NUM_CLASSES), lambda i: (0, 0)),
        ],
        out_specs=pl.BlockSpec((NUM_GRAPHS, NUM_CLASSES), lambda i: (0, 0)),
        out_shape=jax.ShapeDtypeStruct((NUM_GRAPHS, NUM_CLASSES), jnp.float32),
        scratch_shapes=[pltpu.VMEM((NUM_GRAPHS, D), jnp.float32),
                        pltpu.VMEM((NUM_GRAPHS, 1), jnp.float32)],
    )(parts, m3, dinv, b, batch2, mcnt, linw, linb)


@jax.jit
def kernel(x, edge_index, edge_type, batch, emb, W1, b1, W2, b2, W3, b3,
           linW, linb):
    del x, edge_type
    src = edge_index[0]
    dst = edge_index[1]
    pad = EPAD - E
    src_p = jnp.concatenate([src, jnp.full((pad,), N, jnp.int32)]
                            ).reshape(NW, CPT, K)
    dst_p = jnp.concatenate([dst, jnp.full((pad,), N, jnp.int32)]
                            ).reshape(NW, CPT, K)
    padf = NCH * K - E
    pidx = jnp.arange(padf, dtype=jnp.int32)
    src_f = jnp.concatenate([src, pidx % 8192]).reshape(NCH, K)
    dst_f = jnp.concatenate([dst, N + (pidx % (NACC - N))]).reshape(NCH, K)
    col = jnp.arange(D, dtype=jnp.int32)[None, :]
    ones_a = jnp.broadcast_to((col == 0).astype(jnp.float32), (K, D))
    zeros_w = jnp.zeros((STRIPE, D), jnp.float32)

    cnt_s = _sc_count1(src_p, ones_a, zeros_w)
    cnt_d = _sc_count1(dst_p, ones_a, zeros_w)
    csrc = cnt_s[0, :N, 0:1] + cnt_s[1, :N, 0:1]
    cdst = cnt_d[0, :N, 0:1] + cnt_d[1, :N, 0:1]
    dinv = lax.rsqrt(cdst + 1.0)
    mcnt = csrc + cdst

    m1, mt1 = _tc_first(emb, W1, dinv)
    parts1 = _sc_segsum(mt1, src_f, dst_f)
    m2, mt2 = _tc_mid(parts1, m1, dinv, b1.reshape(1, D), W2)
    parts2 = _sc_segsum(mt2, src_f, dst_f)
    m3, mt3 = _tc_mid(parts2, m2, dinv, b2.reshape(1, D), W3)
    parts3 = _sc_segsum(mt3, src_f, dst_f)
    return _tc_pool(parts3, m3, dinv, b3.reshape(1, D),
                    batch.reshape(N, 1), mcnt,
                    linW, linb.reshape(1, NUM_CLASSES))
